# baseline (speedup 1.0000x reference)
import functools

import jax
import jax.numpy as jnp
from jax import lax
from jax.experimental import pallas as pl
from jax.experimental.pallas import tpu as pltpu
from jax.experimental.pallas import tpu_sc as plsc

D_MODEL = 1024
BATCH = 4
SEQ = 8192
B_TOTAL = BATCH * SEQ
NUM_CORES = 2
NUM_SUBCORES = 16
NW = NUM_CORES * NUM_SUBCORES
B_PER_W = B_TOTAL // NW
CHUNK = 16
NBUF = 4
PF = 2
N_CHUNKS = B_PER_W // CHUNK
N_GROUPS = N_CHUNKS // NBUF

_mesh = plsc.VectorSubcoreMesh(core_axis_name="c", subcore_axis_name="s")


@functools.partial(
    pl.kernel,
    mesh=_mesh,
    out_type=jax.ShapeDtypeStruct((B_TOTAL, D_MODEL), jnp.float32),
    scratch_types=[
        pltpu.VMEM((B_PER_W,), jnp.int32),
        pltpu.VMEM((NBUF, CHUNK, D_MODEL), jnp.float32),
        [pltpu.SemaphoreType.DMA] * NBUF,
        [pltpu.SemaphoreType.DMA] * NBUF,
    ],
)
def _emb_gather(x_hbm, table_hbm, out_hbm, idx_v, bufs, gsems, wsems):
    wid = lax.axis_index("s") * NUM_CORES + lax.axis_index("c")
    base = wid * B_PER_W
    pltpu.sync_copy(x_hbm.at[pl.ds(base, B_PER_W)], idx_v)

    def gather(i, b):
        off = pl.multiple_of(i * CHUNK, CHUNK)
        return pltpu.make_async_copy(
            table_hbm.at[idx_v.at[pl.ds(off, CHUNK)]], bufs.at[b], gsems[b]
        )

    def write(i, b):
        off = pl.multiple_of(base + i * CHUNK, CHUNK)
        return pltpu.make_async_copy(
            bufs.at[b], out_hbm.at[pl.ds(off, CHUNK)], wsems[b]
        )

    for b in range(PF):
        gather(b, b).start()

    def body(j, _):
        for b in range(NBUF):
            i = j * NBUF + b
            bp = (b + PF) % NBUF

            @pl.when(i + PF < N_CHUNKS)
            def _():
                @pl.when(i + PF >= NBUF)
                def _():
                    write(i + PF - NBUF, bp).wait()

                gather(i + PF, bp).start()

            gather(i, b).wait()
            write(i, b).start()

        return 0

    lax.fori_loop(0, N_GROUPS, body, 0)

    for i in range(N_CHUNKS - NBUF, N_CHUNKS):
        write(i, i % NBUF).wait()


def kernel(x, table):
    out = _emb_gather(x.reshape(B_TOTAL), table)
    return out.reshape(BATCH, SEQ, D_MODEL)


# speedup vs baseline: 1.0058x; 1.0058x over previous
import functools

import jax
import jax.numpy as jnp
from jax import lax
from jax.experimental import pallas as pl
from jax.experimental.pallas import tpu as pltpu
from jax.experimental.pallas import tpu_sc as plsc

D_MODEL = 1024
BATCH = 4
SEQ = 8192
B_TOTAL = BATCH * SEQ
NUM_CORES = 2
NUM_SUBCORES = 16
NW = NUM_CORES * NUM_SUBCORES
B_PER_W = B_TOTAL // NW
CHUNK = 8
NBUF = 8
PF = 4
N_CHUNKS = B_PER_W // CHUNK
N_GROUPS = N_CHUNKS // NBUF

_mesh = plsc.VectorSubcoreMesh(core_axis_name="c", subcore_axis_name="s")


@functools.partial(
    pl.kernel,
    mesh=_mesh,
    out_type=jax.ShapeDtypeStruct((B_TOTAL, D_MODEL), jnp.float32),
    scratch_types=[
        pltpu.VMEM((B_PER_W,), jnp.int32),
        pltpu.VMEM((NBUF, CHUNK, D_MODEL), jnp.float32),
        [pltpu.SemaphoreType.DMA] * NBUF,
        [pltpu.SemaphoreType.DMA] * NBUF,
    ],
)
def _emb_gather(x_hbm, table_hbm, out_hbm, idx_v, bufs, gsems, wsems):
    wid = lax.axis_index("s") * NUM_CORES + lax.axis_index("c")
    base = wid * B_PER_W
    pltpu.sync_copy(x_hbm.at[pl.ds(base, B_PER_W)], idx_v)

    def gather(i, b):
        off = pl.multiple_of(i * CHUNK, CHUNK)
        return pltpu.make_async_copy(
            table_hbm.at[idx_v.at[pl.ds(off, CHUNK)]], bufs.at[b], gsems[b]
        )

    def write(i, b):
        off = pl.multiple_of(base + i * CHUNK, CHUNK)
        return pltpu.make_async_copy(
            bufs.at[b], out_hbm.at[pl.ds(off, CHUNK)], wsems[b]
        )

    for b in range(PF):
        gather(b, b).start()

    def body(j, _):
        for b in range(NBUF):
            i = j * NBUF + b
            bp = (b + PF) % NBUF

            @pl.when(i + PF < N_CHUNKS)
            def _():
                @pl.when(i + PF >= NBUF)
                def _():
                    write(i + PF - NBUF, bp).wait()

                gather(i + PF, bp).start()

            gather(i, b).wait()
            write(i, b).start()

        return 0

    lax.fori_loop(0, N_GROUPS, body, 0)

    for i in range(N_CHUNKS - NBUF, N_CHUNKS):
        write(i, i % NBUF).wait()


def kernel(x, table):
    out = _emb_gather(x.reshape(B_TOTAL), table)
    return out.reshape(BATCH, SEQ, D_MODEL)


# speedup vs baseline: 1.0066x; 1.0008x over previous
import functools

import jax
import jax.numpy as jnp
from jax import lax
from jax.experimental import pallas as pl
from jax.experimental.pallas import tpu as pltpu
from jax.experimental.pallas import tpu_sc as plsc

D_MODEL = 1024
BATCH = 4
SEQ = 8192
B_TOTAL = BATCH * SEQ
NUM_CORES = 2
NUM_SUBCORES = 16
NW = NUM_CORES * NUM_SUBCORES
B_PER_W = B_TOTAL // NW
CHUNK = 8
NBUF = 8
PF = 6
N_CHUNKS = B_PER_W // CHUNK
N_GROUPS = N_CHUNKS // NBUF

_mesh = plsc.VectorSubcoreMesh(core_axis_name="c", subcore_axis_name="s")


@functools.partial(
    pl.kernel,
    mesh=_mesh,
    out_type=jax.ShapeDtypeStruct((B_TOTAL, D_MODEL), jnp.float32),
    scratch_types=[
        pltpu.VMEM((B_PER_W,), jnp.int32),
        pltpu.VMEM((NBUF, CHUNK, D_MODEL), jnp.float32),
        [pltpu.SemaphoreType.DMA] * NBUF,
        [pltpu.SemaphoreType.DMA] * NBUF,
    ],
)
def _emb_gather(x_hbm, table_hbm, out_hbm, idx_v, bufs, gsems, wsems):
    wid = lax.axis_index("s") * NUM_CORES + lax.axis_index("c")
    base = wid * B_PER_W
    pltpu.sync_copy(x_hbm.at[pl.ds(base, B_PER_W)], idx_v)

    def gather(i, b):
        off = pl.multiple_of(i * CHUNK, CHUNK)
        return pltpu.make_async_copy(
            table_hbm.at[idx_v.at[pl.ds(off, CHUNK)]], bufs.at[b], gsems[b]
        )

    def write(i, b):
        off = pl.multiple_of(base + i * CHUNK, CHUNK)
        return pltpu.make_async_copy(
            bufs.at[b], out_hbm.at[pl.ds(off, CHUNK)], wsems[b]
        )

    for b in range(PF):
        gather(b, b).start()

    def body(j, _):
        for b in range(NBUF):
            i = j * NBUF + b
            bp = (b + PF) % NBUF

            @pl.when(i + PF < N_CHUNKS)
            def _():
                @pl.when(i + PF >= NBUF)
                def _():
                    write(i + PF - NBUF, bp).wait()

                gather(i + PF, bp).start()

            gather(i, b).wait()
            write(i, b).start()

        return 0

    lax.fori_loop(0, N_GROUPS, body, 0)

    for i in range(N_CHUNKS - NBUF, N_CHUNKS):
        write(i, i % NBUF).wait()


def kernel(x, table):
    out = _emb_gather(x.reshape(B_TOTAL), table)
    return out.reshape(BATCH, SEQ, D_MODEL)
